# Initial kernel scaffold; baseline (speedup 1.0000x reference)
import jax
import jax.numpy as jnp
from jax.experimental import pallas as pl


def kernel(h, edge_index, coord, W_e1, b_e1, W_e2, b_e2, W_n1, b_n1, W_n2, b_n2, W_c1, b_c1, W_c2, W_v1, b_v1, W_v2, b_v2):
    raise NotImplementedError("write your pallas kernel here")


# speedup vs baseline: 2.1678x; 2.1678x over previous
import functools

import jax
import jax.numpy as jnp
from jax import lax
from jax.experimental import pallas as pl
from jax.experimental.pallas import tpu as pltpu
from jax.experimental.pallas import tpu_sc as plsc

N = 10000
E = 160000
INF = 256
HNF = 512
ONF = 256

NC = 2
NS = 16
NW = NC * NS
CH = 128
E_PAD = 163840
N_PAD = 10240
PAD_DST = 10200

NCH_G = E_PAD // NW // CH
NCH_S = E_PAD // NS // CH
DRAIN = N_PAD // NS

_mesh = plsc.VectorSubcoreMesh(core_axis_name="c", subcore_axis_name="s",
                               num_cores=NC)


HC = INF + 128


@functools.partial(
    pl.kernel,
    out_type=(
        jax.ShapeDtypeStruct((E_PAD, HC), jnp.float32),
        jax.ShapeDtypeStruct((E_PAD, HC), jnp.float32),
    ),
    mesh=_mesh,
    scratch_types=[
        pltpu.VMEM((CH,), jnp.int32),
        pltpu.VMEM((CH,), jnp.int32),
        pltpu.VMEM((CH, HC), jnp.float32),
        pltpu.VMEM((CH, HC), jnp.float32),
        pltpu.SemaphoreType.DMA,
        pltpu.SemaphoreType.DMA,
    ],
)
def _gather_k(row_h, col_h, hc_h, hrow_o, hcol_o,
              idr, idc, hbr, hbc, s0, s1):
    wid = lax.axis_index("s") * NC + lax.axis_index("c")

    def body(j, _):
        base = (wid * NCH_G + j) * CH
        pltpu.sync_copy(row_h.at[pl.ds(base, CH)], idr)
        pltpu.sync_copy(col_h.at[pl.ds(base, CH)], idc)
        a = pltpu.async_copy(hc_h.at[idr], hbr, s0)
        b = pltpu.async_copy(hc_h.at[idc], hbc, s1)
        a.wait()
        pltpu.sync_copy(hbr, hrow_o.at[pl.ds(base, CH)])
        b.wait()
        pltpu.sync_copy(hbc, hcol_o.at[pl.ds(base, CH)])
        return _

    lax.fori_loop(0, NCH_G, body, None)


@functools.partial(
    pl.kernel,
    out_type=(
        jax.ShapeDtypeStruct((N_PAD, HNF), jnp.float32),
        jax.ShapeDtypeStruct((2 * N_PAD, 128), jnp.float32),
    ),
    mesh=_mesh,
    scratch_types=[
        pltpu.VMEM_SHARED((N_PAD, 128), jnp.float32),
        pltpu.VMEM((CH, 128), jnp.float32),
        pltpu.VMEM((CH,), jnp.int32),
    ],
)
def _scatter_k(idx_h, ef_h, aux_h, zz_h, agg_o, facs_o, acc, ebuf, idxv):
    cid = lax.axis_index("c")
    sid = lax.axis_index("s")

    for half in range(2):
        col0 = (cid * 2 + half) * 128
        pltpu.sync_copy(zz_h, acc.at[pl.ds(sid * DRAIN, DRAIN)])
        plsc.subcore_barrier()

        def chunk(j, _):
            g = sid * NCH_S + j
            pltpu.sync_copy(idx_h.at[g], idxv)
            pltpu.sync_copy(ef_h.at[pl.ds(g * CH, CH), pl.ds(col0, 128)],
                            ebuf)
            pltpu.sync_copy(ebuf, acc.at[idxv], add=True)
            return _

        lax.fori_loop(0, NCH_S, chunk, None)
        plsc.subcore_barrier()
        pltpu.sync_copy(acc.at[pl.ds(sid * DRAIN, DRAIN)],
                        agg_o.at[pl.ds(sid * DRAIN, DRAIN),
                                 pl.ds(col0, 128)])
        plsc.subcore_barrier()

    pltpu.sync_copy(zz_h, acc.at[pl.ds(sid * DRAIN, DRAIN)])
    plsc.subcore_barrier()

    def achunk(j, _):
        g = (sid * NC + cid) * NCH_G + j
        pltpu.sync_copy(idx_h.at[g], idxv)
        pltpu.sync_copy(aux_h.at[pl.ds(g * CH, CH)], ebuf)
        pltpu.sync_copy(ebuf, acc.at[idxv], add=True)
        return _

    lax.fori_loop(0, NCH_G, achunk, None)
    plsc.subcore_barrier()
    pltpu.sync_copy(acc.at[pl.ds(sid * DRAIN, DRAIN)],
                    facs_o.at[pl.ds(cid * N_PAD + sid * DRAIN, DRAIN)])


BE = 1024


def _edge_body(hcrow, hccol, we1a, we1b, wr, be1, we2, be2,
               wc1, bc1, wc2r, ef_o, aux):
    hrow = hcrow[:, 0:INF]
    hcol = hccol[:, 0:INF]
    d = hcrow[:, INF:INF + 128] - hccol[:, INF:INF + 128]
    radial = jnp.sum(d * d, axis=1, keepdims=True)
    x = jnp.dot(hrow, we1a[...], preferred_element_type=jnp.float32)
    x = x + jnp.dot(hcol, we1b[...], preferred_element_type=jnp.float32)
    x = x + radial * wr[...] + be1[...]
    x = jnp.maximum(x, 0.0)
    x = jnp.dot(x, we2[...], preferred_element_type=jnp.float32) + be2[...]
    ef = jnp.maximum(x, 0.0)
    c = jnp.dot(ef, wc1[...], preferred_element_type=jnp.float32) + bc1[...]
    c = jnp.maximum(c, 0.0)
    s = jnp.sum(c * wc2r[...], axis=1, keepdims=True)
    t = jnp.clip(d * s, -100.0, 100.0)
    lane = lax.broadcasted_iota(jnp.int32, t.shape, 1)
    aux[...] = jnp.where(lane == 3, 1.0, t)
    ef_o[...] = ef


def _edge_call(hcrow, hccol, we1a, we1b, wr, be1, we2, be2,
               wc1, bc1, wc2r):
    nb = E_PAD // BE
    full = lambda shape: pl.BlockSpec(shape, lambda i: (0, 0))
    return pl.pallas_call(
        _edge_body,
        grid=(nb,),
        in_specs=[
            pl.BlockSpec((BE, HC), lambda i: (i, 0)),
            pl.BlockSpec((BE, HC), lambda i: (i, 0)),
            full((INF, HNF)), full((INF, HNF)), full((1, HNF)), full((1, HNF)),
            full((HNF, HNF)), full((1, HNF)),
            full((HNF, HNF)), full((1, HNF)), full((1, HNF)),
        ],
        out_specs=[
            pl.BlockSpec((BE, HNF), lambda i: (i, 0)),
            pl.BlockSpec((BE, 128), lambda i: (i, 0)),
        ],
        out_shape=[
            jax.ShapeDtypeStruct((E_PAD, HNF), jnp.float32),
            jax.ShapeDtypeStruct((E_PAD, 128), jnp.float32),
        ],
    )(hcrow, hccol, we1a, we1b, wr, be1, we2, be2, wc1, bc1, wc2r)


BN = 512


def _node_body(hp, agg, fac0, fac1, wn1a, wn1b, bn1,
               wn2, bn2, wv1, bv1, wv2r, bv2r, nout, vel8, f16):
    h = hp[...]
    acc = jnp.dot(h, wn1a[...], preferred_element_type=jnp.float32)
    acc = acc + jnp.dot(agg[...], wn1b[...],
                        preferred_element_type=jnp.float32)
    n1 = jnp.maximum(acc + bn1[...], 0.0)
    nout[...] = jnp.dot(n1, wn2[...], preferred_element_type=jnp.float32) \
        + bn2[...]
    v1 = jnp.maximum(jnp.dot(h, wv1[...], preferred_element_type=jnp.float32)
                     + bv1[...], 0.0)
    vel = jnp.sum(v1 * wv2r[...], axis=1, keepdims=True)
    vel8[...] = jnp.broadcast_to(vel, (BN, 8)) + bv2r[...]
    f = fac0[...] + fac1[...]
    cnt = jnp.maximum(f[:, 3:4], 1.0)
    f16[...] = f[:, 0:16] * (1.0 / cnt)


def _node_call(hp, agg, facs, wn1a, wn1b, bn1,
               wn2, bn2, wv1, bv1, wv2r, bv2r):
    nb = N_PAD // BN
    full = lambda shape: pl.BlockSpec(shape, lambda i: (0, 0))
    return pl.pallas_call(
        _node_body,
        grid=(nb,),
        in_specs=[
            pl.BlockSpec((BN, INF), lambda i: (i, 0)),
            pl.BlockSpec((BN, HNF), lambda i: (i, 0)),
            pl.BlockSpec((BN, 128), lambda i: (i, 0)),
            pl.BlockSpec((BN, 128), lambda i: (nb + i, 0)),
            full((INF, HNF)), full((HNF, HNF)), full((1, HNF)),
            full((HNF, ONF)), full((1, ONF)),
            full((INF, HNF)), full((1, HNF)), full((1, HNF)), full((1, 8)),
        ],
        out_specs=[
            pl.BlockSpec((BN, ONF), lambda i: (i, 0)),
            pl.BlockSpec((BN, 8), lambda i: (i, 0)),
            pl.BlockSpec((BN, 16), lambda i: (i, 0)),
        ],
        out_shape=[
            jax.ShapeDtypeStruct((N_PAD, ONF), jnp.float32),
            jax.ShapeDtypeStruct((N_PAD, 8), jnp.float32),
            jax.ShapeDtypeStruct((N_PAD, 16), jnp.float32),
        ],
    )(hp, agg, facs, facs, wn1a, wn1b, bn1, wn2, bn2,
      wv1, bv1, wv2r, bv2r)


@jax.jit
def kernel(h, edge_index, coord, W_e1, b_e1, W_e2, b_e2, W_n1, b_n1,
           W_n2, b_n2, W_c1, b_c1, W_c2, W_v1, b_v1, W_v2, b_v2):
    row = edge_index[0].astype(jnp.int32)
    col = edge_index[1].astype(jnp.int32)
    pad = E_PAD - E
    rowg = jnp.concatenate([row, jnp.zeros((pad,), jnp.int32)])
    colg = jnp.concatenate([col, jnp.zeros((pad,), jnp.int32)])
    rows = jnp.concatenate([row, jnp.full((pad,), PAD_DST, jnp.int32)])
    idx2d = rows.reshape(E_PAD // CH, CH)
    hc = jnp.concatenate(
        [h, coord, jnp.zeros((N, HC - INF - 3), jnp.float32)], axis=1)
    hp = jnp.pad(h, ((0, N_PAD - N), (0, 0)))

    we1a = W_e1[:INF]
    we1b = W_e1[INF:2 * INF]
    wr = W_e1[2 * INF:].reshape(1, HNF)
    wc2r = W_c2.reshape(1, HNF)
    wv2r = W_v2.reshape(1, HNF)
    wn1a = W_n1[:INF]
    wn1b = W_n1[INF:]
    bv2r = jnp.broadcast_to(b_v2.reshape(1, 1), (1, 8))

    zz = jnp.zeros((DRAIN, 128), jnp.float32)

    hcrow, hccol = _gather_k(rowg, colg, hc)
    ef, aux = _edge_call(
        hcrow, hccol, we1a, we1b, wr, b_e1.reshape(1, HNF),
        W_e2, b_e2.reshape(1, HNF), W_c1, b_c1.reshape(1, HNF), wc2r)
    agg, facs = _scatter_k(idx2d, ef, aux, zz)
    nout, vel8, f16 = _node_call(
        hp, agg, facs, wn1a, wn1b,
        b_n1.reshape(1, HNF), W_n2, b_n2.reshape(1, ONF),
        W_v1, b_v1.reshape(1, HNF), wv2r, bv2r)

    vel = vel8[:N, :1]
    force = f16[:N, :3]
    node_out = nout[:N]
    return (vel, force, node_out)


# speedup vs baseline: 2.5643x; 1.1829x over previous
import functools

import jax
import jax.numpy as jnp
from jax import lax
from jax.experimental import pallas as pl
from jax.experimental.pallas import tpu as pltpu
from jax.experimental.pallas import tpu_sc as plsc

N = 10000
E = 160000
INF = 256
HNF = 512
ONF = 256

NC = 2
NS = 16
NW = NC * NS
CH = 128
E_PAD = 163840
N_PAD = 10240
PAD_DST = 10200

NCH_G = E_PAD // NW // CH
NCH_S = E_PAD // NS // CH
DRAIN = N_PAD // NS

_mesh = plsc.VectorSubcoreMesh(core_axis_name="c", subcore_axis_name="s",
                               num_cores=NC)


HC = INF + 128


@functools.partial(
    pl.kernel,
    out_type=(
        jax.ShapeDtypeStruct((E_PAD, HC), jnp.float32),
        jax.ShapeDtypeStruct((E_PAD, HC), jnp.float32),
    ),
    mesh=_mesh,
    scratch_types=[
        pltpu.VMEM((E_PAD // NW,), jnp.int32),
        pltpu.VMEM((CH, HC), jnp.float32),
        pltpu.VMEM((CH, HC), jnp.float32),
        pltpu.SemaphoreType.DMA,
        pltpu.SemaphoreType.DMA,
    ],
)
def _gather_k(row_h, col_h, hc_h, hrow_o, hcol_o, idall, b0, b1, s0, s1):
    wid = lax.axis_index("s") * NC + lax.axis_index("c")
    epw = E_PAD // NW
    base = wid * epw

    def side(idx_h, out_h):
        pltpu.sync_copy(idx_h.at[pl.ds(base, epw)], idall)

        def ix(c):
            return idall.at[pl.ds(c * CH, CH)]

        pltpu.async_copy(hc_h.at[ix(0)], b0, s0)

        def body(jj, _):
            c0 = 2 * jj
            c1 = c0 + 1
            c2 = jnp.minimum(c0 + 2, NCH_G - 1)
            pltpu.async_copy(hc_h.at[ix(c1)], b1, s1)
            pltpu.make_async_copy(hc_h.at[pl.ds(0, CH)], b0, s0).wait()
            pltpu.sync_copy(b0, out_h.at[pl.ds(base + c0 * CH, CH)])
            pltpu.async_copy(hc_h.at[ix(c2)], b0, s0)
            pltpu.make_async_copy(hc_h.at[pl.ds(0, CH)], b1, s1).wait()
            pltpu.sync_copy(b1, out_h.at[pl.ds(base + c1 * CH, CH)])
            return _

        lax.fori_loop(0, NCH_G // 2, body, None)
        pltpu.make_async_copy(hc_h.at[pl.ds(0, CH)], b0, s0).wait()

    side(row_h, hrow_o)
    side(col_h, hcol_o)


@functools.partial(
    pl.kernel,
    out_type=(
        jax.ShapeDtypeStruct((N_PAD, HNF), jnp.float32),
        jax.ShapeDtypeStruct((2 * N_PAD, 128), jnp.float32),
    ),
    mesh=_mesh,
    scratch_types=[
        pltpu.VMEM_SHARED((N_PAD, 128), jnp.float32),
        pltpu.VMEM((CH, 128), jnp.float32),
        pltpu.VMEM((CH, 128), jnp.float32),
        pltpu.VMEM((CH,), jnp.int32),
        pltpu.VMEM((CH,), jnp.int32),
        pltpu.SemaphoreType.DMA,
        pltpu.SemaphoreType.DMA,
    ],
)
def _scatter_k(idx_h, ef_h, aux_h, zz_h, agg_o, facs_o,
               acc, eb0, eb1, iv0, iv1, s0, s1):
    cid = lax.axis_index("c")
    sid = lax.axis_index("s")
    dummy = aux_h.at[pl.ds(0, CH)]

    def scatter_phase(src_fn, g_base, n_chunks):
        pltpu.sync_copy(idx_h.at[g_base], iv0)
        pltpu.async_copy(src_fn(g_base), eb0, s0)

        def body(jj, _):
            c0 = g_base + 2 * jj
            c1 = c0 + 1
            c2 = jnp.minimum(c0 + 2, g_base + n_chunks - 1)
            pltpu.sync_copy(idx_h.at[c1], iv1)
            pltpu.async_copy(src_fn(c1), eb1, s1)
            pltpu.make_async_copy(dummy, eb0, s0).wait()
            pltpu.sync_copy(eb0, acc.at[iv0], add=True)
            pltpu.sync_copy(idx_h.at[c2], iv0)
            pltpu.async_copy(src_fn(c2), eb0, s0)
            pltpu.make_async_copy(dummy, eb1, s1).wait()
            pltpu.sync_copy(eb1, acc.at[iv1], add=True)
            return _

        lax.fori_loop(0, n_chunks // 2, body, None)
        pltpu.make_async_copy(dummy, eb0, s0).wait()

    for half in range(2):
        col0 = (cid * 2 + half) * 128
        pltpu.sync_copy(zz_h, acc.at[pl.ds(sid * DRAIN, DRAIN)])
        plsc.subcore_barrier()
        scatter_phase(
            lambda c: ef_h.at[pl.ds(c * CH, CH), pl.ds(col0, 128)],
            sid * NCH_S, NCH_S)
        plsc.subcore_barrier()
        pltpu.sync_copy(acc.at[pl.ds(sid * DRAIN, DRAIN)],
                        agg_o.at[pl.ds(sid * DRAIN, DRAIN),
                                 pl.ds(col0, 128)])
        plsc.subcore_barrier()

    pltpu.sync_copy(zz_h, acc.at[pl.ds(sid * DRAIN, DRAIN)])
    plsc.subcore_barrier()
    scatter_phase(lambda c: aux_h.at[pl.ds(c * CH, CH)],
                  (sid * NC + cid) * NCH_G, NCH_G)
    plsc.subcore_barrier()
    pltpu.sync_copy(acc.at[pl.ds(sid * DRAIN, DRAIN)],
                    facs_o.at[pl.ds(cid * N_PAD + sid * DRAIN, DRAIN)])


BE = 1024


def _edge_body(hcrow, hccol, we1a, we1b, wr, be1, we2, be2,
               wc1, bc1, wc2r, ef_o, aux):
    hrow = hcrow[:, 0:INF]
    hcol = hccol[:, 0:INF]
    d = hcrow[:, INF:INF + 128] - hccol[:, INF:INF + 128]
    radial = jnp.sum(d * d, axis=1, keepdims=True)
    x = jnp.dot(hrow, we1a[...], preferred_element_type=jnp.float32)
    x = x + jnp.dot(hcol, we1b[...], preferred_element_type=jnp.float32)
    x = x + radial * wr[...] + be1[...]
    x = jnp.maximum(x, 0.0)
    x = jnp.dot(x, we2[...], preferred_element_type=jnp.float32) + be2[...]
    ef = jnp.maximum(x, 0.0)
    c = jnp.dot(ef, wc1[...], preferred_element_type=jnp.float32) + bc1[...]
    c = jnp.maximum(c, 0.0)
    s = jnp.sum(c * wc2r[...], axis=1, keepdims=True)
    t = jnp.clip(d * s, -100.0, 100.0)
    lane = lax.broadcasted_iota(jnp.int32, t.shape, 1)
    aux[...] = jnp.where(lane == 3, 1.0, t)
    ef_o[...] = ef


def _edge_call(hcrow, hccol, we1a, we1b, wr, be1, we2, be2,
               wc1, bc1, wc2r):
    nb = E_PAD // BE
    full = lambda shape: pl.BlockSpec(shape, lambda i: (0, 0))
    return pl.pallas_call(
        _edge_body,
        grid=(nb,),
        in_specs=[
            pl.BlockSpec((BE, HC), lambda i: (i, 0)),
            pl.BlockSpec((BE, HC), lambda i: (i, 0)),
            full((INF, HNF)), full((INF, HNF)), full((1, HNF)), full((1, HNF)),
            full((HNF, HNF)), full((1, HNF)),
            full((HNF, HNF)), full((1, HNF)), full((1, HNF)),
        ],
        out_specs=[
            pl.BlockSpec((BE, HNF), lambda i: (i, 0)),
            pl.BlockSpec((BE, 128), lambda i: (i, 0)),
        ],
        out_shape=[
            jax.ShapeDtypeStruct((E_PAD, HNF), jnp.float32),
            jax.ShapeDtypeStruct((E_PAD, 128), jnp.float32),
        ],
    )(hcrow, hccol, we1a, we1b, wr, be1, we2, be2, wc1, bc1, wc2r)


BN = 512


def _node_body(hp, agg, fac0, fac1, wn1a, wn1b, bn1,
               wn2, bn2, wv1, bv1, wv2r, bv2r, nout, vel8, f16):
    h = hp[...]
    acc = jnp.dot(h, wn1a[...], preferred_element_type=jnp.float32)
    acc = acc + jnp.dot(agg[...], wn1b[...],
                        preferred_element_type=jnp.float32)
    n1 = jnp.maximum(acc + bn1[...], 0.0)
    nout[...] = jnp.dot(n1, wn2[...], preferred_element_type=jnp.float32) \
        + bn2[...]
    v1 = jnp.maximum(jnp.dot(h, wv1[...], preferred_element_type=jnp.float32)
                     + bv1[...], 0.0)
    vel = jnp.sum(v1 * wv2r[...], axis=1, keepdims=True)
    vel8[...] = jnp.broadcast_to(vel, (BN, 8)) + bv2r[...]
    f = fac0[...] + fac1[...]
    cnt = jnp.maximum(f[:, 3:4], 1.0)
    f16[...] = f[:, 0:16] * (1.0 / cnt)


def _node_call(hp, agg, facs, wn1a, wn1b, bn1,
               wn2, bn2, wv1, bv1, wv2r, bv2r):
    nb = N_PAD // BN
    full = lambda shape: pl.BlockSpec(shape, lambda i: (0, 0))
    return pl.pallas_call(
        _node_body,
        grid=(nb,),
        in_specs=[
            pl.BlockSpec((BN, INF), lambda i: (i, 0)),
            pl.BlockSpec((BN, HNF), lambda i: (i, 0)),
            pl.BlockSpec((BN, 128), lambda i: (i, 0)),
            pl.BlockSpec((BN, 128), lambda i: (nb + i, 0)),
            full((INF, HNF)), full((HNF, HNF)), full((1, HNF)),
            full((HNF, ONF)), full((1, ONF)),
            full((INF, HNF)), full((1, HNF)), full((1, HNF)), full((1, 8)),
        ],
        out_specs=[
            pl.BlockSpec((BN, ONF), lambda i: (i, 0)),
            pl.BlockSpec((BN, 8), lambda i: (i, 0)),
            pl.BlockSpec((BN, 16), lambda i: (i, 0)),
        ],
        out_shape=[
            jax.ShapeDtypeStruct((N_PAD, ONF), jnp.float32),
            jax.ShapeDtypeStruct((N_PAD, 8), jnp.float32),
            jax.ShapeDtypeStruct((N_PAD, 16), jnp.float32),
        ],
    )(hp, agg, facs, facs, wn1a, wn1b, bn1, wn2, bn2,
      wv1, bv1, wv2r, bv2r)


@jax.jit
def kernel(h, edge_index, coord, W_e1, b_e1, W_e2, b_e2, W_n1, b_n1,
           W_n2, b_n2, W_c1, b_c1, W_c2, W_v1, b_v1, W_v2, b_v2):
    row = edge_index[0].astype(jnp.int32)
    col = edge_index[1].astype(jnp.int32)
    pad = E_PAD - E
    rowg = jnp.concatenate([row, jnp.zeros((pad,), jnp.int32)])
    colg = jnp.concatenate([col, jnp.zeros((pad,), jnp.int32)])
    rows = jnp.concatenate([row, jnp.full((pad,), PAD_DST, jnp.int32)])
    idx2d = rows.reshape(E_PAD // CH, CH)
    hc = jnp.concatenate(
        [h, coord, jnp.zeros((N, HC - INF - 3), jnp.float32)], axis=1)
    hp = jnp.pad(h, ((0, N_PAD - N), (0, 0)))

    we1a = W_e1[:INF]
    we1b = W_e1[INF:2 * INF]
    wr = W_e1[2 * INF:].reshape(1, HNF)
    wc2r = W_c2.reshape(1, HNF)
    wv2r = W_v2.reshape(1, HNF)
    wn1a = W_n1[:INF]
    wn1b = W_n1[INF:]
    bv2r = jnp.broadcast_to(b_v2.reshape(1, 1), (1, 8))

    zz = jnp.zeros((DRAIN, 128), jnp.float32)

    hcrow, hccol = _gather_k(rowg, colg, hc)
    ef, aux = _edge_call(
        hcrow, hccol, we1a, we1b, wr, b_e1.reshape(1, HNF),
        W_e2, b_e2.reshape(1, HNF), W_c1, b_c1.reshape(1, HNF), wc2r)
    agg, facs = _scatter_k(idx2d, ef, aux, zz)
    nout, vel8, f16 = _node_call(
        hp, agg, facs, wn1a, wn1b,
        b_n1.reshape(1, HNF), W_n2, b_n2.reshape(1, ONF),
        W_v1, b_v1.reshape(1, HNF), wv2r, bv2r)

    vel = vel8[:N, :1]
    force = f16[:N, :3]
    node_out = nout[:N]
    return (vel, force, node_out)
